# baseline (speedup 1.0000x reference)
import functools

import jax
import jax.numpy as jnp
from jax import lax
from jax.experimental import pallas as pl
from jax.experimental.pallas import tpu as pltpu
from jax.experimental.pallas import tpu_sc as plsc

N = 50000
L = 12500
NB = 4
E = 200000
D = 64
EPS_LN = 1e-5

TBL = 12800
ACTIVE = 12544
SENT = 12700
EPAD = 204800
CH = 128
RPT = TBL // 16

_ROWS = 2000

_mesh = plsc.VectorSubcoreMesh(core_axis_name="c", subcore_axis_name="s",
                               num_cores=2, num_subcores=16)


def _hist_body(dst_hbm, out_hbm, didx, dsh, ones, zeros, hist_sh):
    c = lax.axis_index("c")
    s = lax.axis_index("s")
    wid = s * 2 + c
    one16 = jnp.ones((16,), jnp.int32)
    z16 = jnp.zeros((16,), jnp.int32)
    for i in range(CH // 16):
        ones[pl.ds(16 * i, 16)] = one16

    @pl.loop(0, RPT // 16)
    def _zfill(i):
        zeros[pl.ds(16 * i, 16)] = z16

    pltpu.sync_copy(zeros, hist_sh.at[pl.ds(s * RPT, RPT)])
    plsc.subcore_barrier()

    @pl.loop(0, EPAD // (32 * CH))
    def _chunk(k):
        base = wid * (EPAD // 32) + k * CH
        pltpu.sync_copy(dst_hbm.at[pl.ds(base, CH)], didx)
        for j in range(NB):
            sh = jnp.zeros((16,), jnp.int32) + (10 * j)
            for i in range(CH // 16):
                dsh[pl.ds(16 * i, 16)] = didx[pl.ds(16 * i, 16)] + sh
            pltpu.sync_copy(ones, hist_sh.at[dsh], add=True)

    plsc.subcore_barrier()
    pltpu.sync_copy(hist_sh.at[pl.ds(s * RPT, RPT)], zeros)
    pltpu.sync_copy(zeros, out_hbm.at[pl.ds(c * TBL + s * RPT, RPT)])


_hist = functools.partial(
    pl.kernel,
    out_type=jax.ShapeDtypeStruct((2 * TBL,), jnp.int32),
    mesh=_mesh,
    scratch_types=[
        pltpu.VMEM((CH,), jnp.int32),
        pltpu.VMEM((CH,), jnp.int32),
        pltpu.VMEM((CH,), jnp.int32),
        pltpu.VMEM((RPT,), jnp.int32),
        pltpu.VMEM_SHARED((TBL,), jnp.int32),
    ],
)(_hist_body)


def _dinv_body(hist_ref, dinv_ref):
    deg = jnp.sum(hist_ref[...], axis=0, keepdims=True) + 1
    dinv_ref[...] = lax.rsqrt(deg.astype(jnp.float32))


def _dinv_pass(hists):
    return pl.pallas_call(
        _dinv_body,
        out_shape=jax.ShapeDtypeStruct((1, TBL), jnp.float32),
    )(hists)


def _dense_body(x_ref, dinv_ref, W1_ref, b1_ref, W2_ref, b2_ref, g_ref,
                be_ref, Wg_ref, u_ref, outf_ref):
    x = x_ref[...]
    h = jax.nn.silu(x * W1_ref[...] + b1_ref[...])
    h = jax.nn.silu(
        jnp.dot(h, W2_ref[...], preferred_element_type=jnp.float32)
        + b2_ref[...])
    mu = jnp.mean(h, axis=-1, keepdims=True)
    var = jnp.mean((h - mu) ** 2, axis=-1, keepdims=True)
    hn = (h - mu) * lax.rsqrt(var + EPS_LN) * g_ref[...] + be_ref[...]
    h = jax.nn.silu(hn)
    hw = jnp.dot(h, Wg_ref[...], preferred_element_type=jnp.float32)
    u_ref[...] = dinv_ref[...] * hw
    outf_ref[...] = jax.nn.relu(hw) + hw


def _dense_pass(x, dinv_full, W1, b1, W2, b2, ln_g, ln_b, Wg):
    grid = (N // _ROWS,)
    full = lambda i: (0, 0)
    row_spec = pl.BlockSpec((_ROWS, 1), lambda i: (i, 0))
    out_spec = pl.BlockSpec((_ROWS, D), lambda i: (i, 0))
    w_spec = pl.BlockSpec((1, D), full)
    m_spec = pl.BlockSpec((D, D), full)
    return pl.pallas_call(
        _dense_body,
        grid=grid,
        in_specs=[row_spec, row_spec, w_spec, w_spec, m_spec, w_spec,
                  w_spec, w_spec, m_spec],
        out_specs=[out_spec, out_spec],
        out_shape=[jax.ShapeDtypeStruct((N, D), jnp.float32),
                   jax.ShapeDtypeStruct((N, D), jnp.float32)],
    )(x, dinv_full, W1, b1.reshape(1, D), W2, b2.reshape(1, D),
      ln_g.reshape(1, D), ln_b.reshape(1, D), Wg)


BCH = 256
NG = EPAD // (16 * BCH)
GR = 2 * BCH
SB = 5
NSB = NG // SB


def _edge_scatter_body(srcp_hbm, dstp_hbm, u_hbm, out_hbm, sbig, dbig,
                       sidx0, didx0, sidx1, didx1, rows0, rows1, acc_sh,
                       sem0, sem1):
    c = lax.axis_index("c")
    s = lax.axis_index("s")
    z16 = jnp.zeros((16,), jnp.float32)
    sh0 = jnp.zeros((16,), jnp.int32) + c * 20
    sh1 = jnp.zeros((16,), jnp.int32) + (c * 20 + 10)
    ebase = s * (EPAD // 16)

    @pl.loop(0, GR)
    def _zrow(i):
        for k in range(D // 16):
            rows0[i, pl.ds(k * 16, 16)] = z16

    pltpu.sync_copy(rows0, acc_sh.at[pl.ds(s * RPT, GR)])
    pltpu.sync_copy(rows0.at[pl.ds(0, RPT - GR)],
                    acc_sh.at[pl.ds(s * RPT + GR, RPT - GR)])
    plsc.subcore_barrier()

    def _fire(k, sidx, didx, rows, sem):
        for i in range(BCH // 16):
            vs = sbig[pl.ds(k * BCH + 16 * i, 16)]
            vd = dbig[pl.ds(k * BCH + 16 * i, 16)]
            sidx[pl.ds(16 * i, 16)] = vs + sh0
            sidx[pl.ds(BCH + 16 * i, 16)] = vs + sh1
            didx[pl.ds(16 * i, 16)] = vd + sh0
            didx[pl.ds(BCH + 16 * i, 16)] = vd + sh1
        pltpu.async_copy(u_hbm.at[sidx], rows, sem)

    def _drain_scatter(didx, rows, sem):
        pltpu.make_async_copy(u_hbm.at[pl.ds(0, GR)], rows, sem).wait()
        pltpu.sync_copy(rows, acc_sh.at[didx], add=True)

    ring = [(sidx0, didx0, rows0, sem0), (sidx1, didx1, rows1, sem1)]

    @pl.loop(0, NSB)
    def _sblk(u):
        base = ebase + u * (SB * BCH)
        pltpu.sync_copy(srcp_hbm.at[pl.ds(base, SB * BCH)], sbig)
        pltpu.sync_copy(dstp_hbm.at[pl.ds(base, SB * BCH)], dbig)
        _fire(0, *ring[0])
        for k in range(1, SB):
            _fire(k, *ring[k % 2])
            _drain_scatter(*ring[(k - 1) % 2][1:])
        _drain_scatter(*ring[(SB - 1) % 2][1:])

    plsc.subcore_barrier()
    pltpu.sync_copy(acc_sh.at[pl.ds(s * RPT, GR)], rows0)
    pltpu.sync_copy(rows0, out_hbm.at[pl.ds(c * TBL + s * RPT, GR)])
    pltpu.sync_copy(acc_sh.at[pl.ds(s * RPT + GR, RPT - GR)],
                    rows0.at[pl.ds(0, RPT - GR)])
    pltpu.sync_copy(rows0.at[pl.ds(0, RPT - GR)],
                    out_hbm.at[pl.ds(c * TBL + s * RPT + GR, RPT - GR)])


_edge_scatter = functools.partial(
    pl.kernel,
    out_type=jax.ShapeDtypeStruct((2 * TBL, D), jnp.float32),
    mesh=_mesh,
    compiler_params=pltpu.CompilerParams(use_tc_tiling_on_sc=False),
    scratch_types=[
        pltpu.VMEM((SB * BCH,), jnp.int32),
        pltpu.VMEM((SB * BCH,), jnp.int32),
        pltpu.VMEM((GR,), jnp.int32),
        pltpu.VMEM((GR,), jnp.int32),
        pltpu.VMEM((GR,), jnp.int32),
        pltpu.VMEM((GR,), jnp.int32),
        pltpu.VMEM((GR, D), jnp.float32),
        pltpu.VMEM((GR, D), jnp.float32),
        pltpu.VMEM_SHARED((TBL, D), jnp.float32),
        pltpu.SemaphoreType.DMA,
        pltpu.SemaphoreType.DMA,
    ],
)(_edge_scatter_body)


def _combine_body(a0_ref, a1_ref, u_ref, dinv_ref, out_ref):
    a = a0_ref[...] + a1_ref[...]
    y = dinv_ref[...] * (u_ref[...] + a)
    out_ref[...] = jax.nn.relu(y) + y


def _combine_pass(u, acc, dinv_full):
    grid = (ACTIVE // 128,)
    return pl.pallas_call(
        _combine_body,
        grid=grid,
        in_specs=[pl.BlockSpec((128, D), lambda i: (i, 0)),
                  pl.BlockSpec((128, D), lambda i: (i + TBL // 128, 0)),
                  pl.BlockSpec((128, D), lambda i: (i, 0)),
                  pl.BlockSpec((128, 1), lambda i: (i, 0))],
        out_specs=pl.BlockSpec((128, D), lambda i: (i, 0)),
        out_shape=jax.ShapeDtypeStruct((ACTIVE, D), jnp.float32),
    )(acc, acc, u, dinv_full)


def kernel(x, go_edge_index, W1, b1, W2, b2, ln_g, ln_b, Wg):
    pad = jnp.full((EPAD - E,), SENT, dtype=jnp.int32)
    srcp = jnp.concatenate([go_edge_index[0], pad])
    dstp = jnp.concatenate([go_edge_index[1], pad])

    hists = _hist(dstp).reshape(2, TBL)
    dinv = _dinv_pass(hists)
    dinv_full = jnp.concatenate(
        [dinv.reshape(TBL, 1),
         jnp.ones((N - TBL, 1), dtype=jnp.float32)], axis=0)

    u, outf = _dense_pass(x, dinv_full, W1, b1, W2, b2, ln_g, ln_b, Wg)
    acc = _edge_scatter(srcp, dstp, u)
    out_active = _combine_pass(u, acc, dinv_full)
    return jnp.concatenate([out_active, outf[ACTIVE:]], axis=0)


# speedup vs baseline: 1.1533x; 1.1533x over previous
import functools

import jax
import jax.numpy as jnp
from jax import lax
from jax.experimental import pallas as pl
from jax.experimental.pallas import tpu as pltpu
from jax.experimental.pallas import tpu_sc as plsc

N = 50000
L = 12500
NB = 4
E = 200000
D = 64
EPS_LN = 1e-5

TBL = 12800
ACTIVE = 12544
SENT = 12700
EPAD = 204800
CH = 128
RPT = TBL // 16

_ROWS = 2000

_mesh = plsc.VectorSubcoreMesh(core_axis_name="c", subcore_axis_name="s",
                               num_cores=2, num_subcores=16)


def _hist_body(dst_hbm, out_hbm, didx, dsh, ones, zeros, hist_sh):
    c = lax.axis_index("c")
    s = lax.axis_index("s")
    wid = s * 2 + c
    one16 = jnp.ones((16,), jnp.int32)
    z16 = jnp.zeros((16,), jnp.int32)
    for i in range(CH // 16):
        ones[pl.ds(16 * i, 16)] = one16

    @pl.loop(0, RPT // 16)
    def _zfill(i):
        zeros[pl.ds(16 * i, 16)] = z16

    pltpu.sync_copy(zeros, hist_sh.at[pl.ds(s * RPT, RPT)])
    plsc.subcore_barrier()

    @pl.loop(0, EPAD // (32 * CH))
    def _chunk(k):
        base = wid * (EPAD // 32) + k * CH
        pltpu.sync_copy(dst_hbm.at[pl.ds(base, CH)], didx)
        for j in range(NB):
            sh = jnp.zeros((16,), jnp.int32) + (10 * j)
            for i in range(CH // 16):
                dsh[pl.ds(16 * i, 16)] = didx[pl.ds(16 * i, 16)] + sh
            pltpu.sync_copy(ones, hist_sh.at[dsh], add=True)

    plsc.subcore_barrier()
    pltpu.sync_copy(hist_sh.at[pl.ds(s * RPT, RPT)], zeros)
    pltpu.sync_copy(zeros, out_hbm.at[pl.ds(c * TBL + s * RPT, RPT)])


_hist = functools.partial(
    pl.kernel,
    out_type=jax.ShapeDtypeStruct((2 * TBL,), jnp.int32),
    mesh=_mesh,
    scratch_types=[
        pltpu.VMEM((CH,), jnp.int32),
        pltpu.VMEM((CH,), jnp.int32),
        pltpu.VMEM((CH,), jnp.int32),
        pltpu.VMEM((RPT,), jnp.int32),
        pltpu.VMEM_SHARED((TBL,), jnp.int32),
    ],
)(_hist_body)


def _dense_body(x_ref, W1_ref, b1_ref, W2_ref, b2_ref, g_ref,
                be_ref, Wg_ref, hw_ref, outf_ref):
    x = x_ref[...]
    h = jax.nn.silu(x * W1_ref[...] + b1_ref[...])
    h = jax.nn.silu(
        jnp.dot(h, W2_ref[...], preferred_element_type=jnp.float32)
        + b2_ref[...])
    mu = jnp.mean(h, axis=-1, keepdims=True)
    var = jnp.mean((h - mu) ** 2, axis=-1, keepdims=True)
    hn = (h - mu) * lax.rsqrt(var + EPS_LN) * g_ref[...] + be_ref[...]
    h = jax.nn.silu(hn)
    hw = jnp.dot(h, Wg_ref[...], preferred_element_type=jnp.float32)
    hw_ref[...] = hw
    outf_ref[...] = jax.nn.relu(hw) + hw


def _dense_pass(x, W1, b1, W2, b2, ln_g, ln_b, Wg):
    grid = (N // _ROWS,)
    full = lambda i: (0, 0)
    row_spec = pl.BlockSpec((_ROWS, 1), lambda i: (i, 0))
    out_spec = pl.BlockSpec((_ROWS, D), lambda i: (i, 0))
    w_spec = pl.BlockSpec((1, D), full)
    m_spec = pl.BlockSpec((D, D), full)
    return pl.pallas_call(
        _dense_body,
        grid=grid,
        in_specs=[row_spec, w_spec, w_spec, m_spec, w_spec,
                  w_spec, w_spec, m_spec],
        out_specs=[out_spec, out_spec],
        out_shape=[jax.ShapeDtypeStruct((N, D), jnp.float32),
                   jax.ShapeDtypeStruct((N, D), jnp.float32)],
    )(x, W1, b1.reshape(1, D), W2, b2.reshape(1, D),
      ln_g.reshape(1, D), ln_b.reshape(1, D), Wg)


_UR = 1600


def _upass_body(h0_ref, h1_ref, hw_ref, u_ref, dinv_ref):
    deg = h0_ref[...] + h1_ref[...] + 1
    dinv = lax.rsqrt(deg.astype(jnp.float32))
    dinv_ref[...] = dinv
    u_ref[...] = dinv * hw_ref[...]


def _u_pass(hists_col, hw):
    grid = (TBL // _UR,)
    col = lambda i: (i, 0)
    return pl.pallas_call(
        _upass_body,
        grid=grid,
        in_specs=[pl.BlockSpec((_UR, 1), col),
                  pl.BlockSpec((_UR, 1), lambda i: (i + TBL // _UR, 0)),
                  pl.BlockSpec((_UR, D), lambda i: (i, 0))],
        out_specs=[pl.BlockSpec((_UR, D), lambda i: (i, 0)),
                   pl.BlockSpec((_UR, 1), col)],
        out_shape=[jax.ShapeDtypeStruct((TBL, D), jnp.float32),
                   jax.ShapeDtypeStruct((TBL, 1), jnp.float32)],
    )(hists_col, hists_col, hw)


BCH = 256
NG = EPAD // (16 * BCH)
GR = 2 * BCH
SB = 5
NSB = NG // SB


def _edge_scatter_body(srcp_hbm, dstp_hbm, u_hbm, out_hbm, sbig, dbig,
                       sidx0, didx0, sidx1, didx1, rows0, rows1, acc_sh,
                       sem0, sem1):
    c = lax.axis_index("c")
    s = lax.axis_index("s")
    z16 = jnp.zeros((16,), jnp.float32)
    sh0 = jnp.zeros((16,), jnp.int32) + c * 20
    sh1 = jnp.zeros((16,), jnp.int32) + (c * 20 + 10)
    ebase = s * (EPAD // 16)

    @pl.loop(0, GR)
    def _zrow(i):
        for k in range(D // 16):
            rows0[i, pl.ds(k * 16, 16)] = z16

    pltpu.sync_copy(rows0, acc_sh.at[pl.ds(s * RPT, GR)])
    pltpu.sync_copy(rows0.at[pl.ds(0, RPT - GR)],
                    acc_sh.at[pl.ds(s * RPT + GR, RPT - GR)])
    plsc.subcore_barrier()

    def _fire(k, sidx, didx, rows, sem):
        for i in range(BCH // 16):
            vs = sbig[pl.ds(k * BCH + 16 * i, 16)]
            vd = dbig[pl.ds(k * BCH + 16 * i, 16)]
            sidx[pl.ds(16 * i, 16)] = vs + sh0
            sidx[pl.ds(BCH + 16 * i, 16)] = vs + sh1
            didx[pl.ds(16 * i, 16)] = vd + sh0
            didx[pl.ds(BCH + 16 * i, 16)] = vd + sh1
        pltpu.async_copy(u_hbm.at[sidx], rows, sem)

    def _drain_scatter(didx, rows, sem):
        pltpu.make_async_copy(u_hbm.at[pl.ds(0, GR)], rows, sem).wait()
        pltpu.sync_copy(rows, acc_sh.at[didx], add=True)

    ring = [(sidx0, didx0, rows0, sem0), (sidx1, didx1, rows1, sem1)]

    @pl.loop(0, NSB)
    def _sblk(u):
        base = ebase + u * (SB * BCH)
        pltpu.sync_copy(srcp_hbm.at[pl.ds(base, SB * BCH)], sbig)
        pltpu.sync_copy(dstp_hbm.at[pl.ds(base, SB * BCH)], dbig)
        _fire(0, *ring[0])
        for k in range(1, SB):
            _fire(k, *ring[k % 2])
            _drain_scatter(*ring[(k - 1) % 2][1:])
        _drain_scatter(*ring[(SB - 1) % 2][1:])

    plsc.subcore_barrier()
    pltpu.sync_copy(acc_sh.at[pl.ds(s * RPT, GR)], rows0)
    pltpu.sync_copy(rows0, out_hbm.at[pl.ds(c * TBL + s * RPT, GR)])
    pltpu.sync_copy(acc_sh.at[pl.ds(s * RPT + GR, RPT - GR)],
                    rows0.at[pl.ds(0, RPT - GR)])
    pltpu.sync_copy(rows0.at[pl.ds(0, RPT - GR)],
                    out_hbm.at[pl.ds(c * TBL + s * RPT + GR, RPT - GR)])


_edge_scatter = functools.partial(
    pl.kernel,
    out_type=jax.ShapeDtypeStruct((2 * TBL, D), jnp.float32),
    mesh=_mesh,
    compiler_params=pltpu.CompilerParams(use_tc_tiling_on_sc=False),
    scratch_types=[
        pltpu.VMEM((SB * BCH,), jnp.int32),
        pltpu.VMEM((SB * BCH,), jnp.int32),
        pltpu.VMEM((GR,), jnp.int32),
        pltpu.VMEM((GR,), jnp.int32),
        pltpu.VMEM((GR,), jnp.int32),
        pltpu.VMEM((GR,), jnp.int32),
        pltpu.VMEM((GR, D), jnp.float32),
        pltpu.VMEM((GR, D), jnp.float32),
        pltpu.VMEM_SHARED((TBL, D), jnp.float32),
        pltpu.SemaphoreType.DMA,
        pltpu.SemaphoreType.DMA,
    ],
)(_edge_scatter_body)


def _combine_body(a0_ref, a1_ref, u_ref, dinv_ref, out_ref):
    a = a0_ref[...] + a1_ref[...]
    y = dinv_ref[...] * (u_ref[...] + a)
    out_ref[...] = jax.nn.relu(y) + y


def _combine_pass(u, acc, dinv):
    grid = (ACTIVE // 256,)
    return pl.pallas_call(
        _combine_body,
        grid=grid,
        in_specs=[pl.BlockSpec((256, D), lambda i: (i, 0)),
                  pl.BlockSpec((256, D), lambda i: (i + TBL // 256, 0)),
                  pl.BlockSpec((256, D), lambda i: (i, 0)),
                  pl.BlockSpec((256, 1), lambda i: (i, 0))],
        out_specs=pl.BlockSpec((256, D), lambda i: (i, 0)),
        out_shape=jax.ShapeDtypeStruct((ACTIVE, D), jnp.float32),
    )(acc, acc, u, dinv)


def kernel(x, go_edge_index, W1, b1, W2, b2, ln_g, ln_b, Wg):
    pad = jnp.full((EPAD - E,), SENT, dtype=jnp.int32)
    srcp = jnp.concatenate([go_edge_index[0], pad])
    dstp = jnp.concatenate([go_edge_index[1], pad])

    hists_col = _hist(dstp).reshape(2 * TBL, 1)
    hw, outf = _dense_pass(x, W1, b1, W2, b2, ln_g, ln_b, Wg)
    u, dinv = _u_pass(hists_col, hw[:TBL])
    acc = _edge_scatter(srcp, dstp, u)
    out_active = _combine_pass(u, acc, dinv)
    return jnp.concatenate([out_active, outf[ACTIVE:]], axis=0)


# speedup vs baseline: 1.2491x; 1.0831x over previous
import functools

import jax
import jax.numpy as jnp
from jax import lax
from jax.experimental import pallas as pl
from jax.experimental.pallas import tpu as pltpu
from jax.experimental.pallas import tpu_sc as plsc

N = 50000
L = 12500
NB = 4
E = 200000
D = 64
EPS_LN = 1e-5

TBL = 12800
ACTIVE = 12544
SENT = 12700
EPAD = 204800
CH = 128
RPT = TBL // 16

_ROWS = 2000

_mesh = plsc.VectorSubcoreMesh(core_axis_name="c", subcore_axis_name="s",
                               num_cores=2, num_subcores=16)


def _hist_body(dst_hbm, out_hbm, didx, dsh, ones, zeros, hist_sh):
    c = lax.axis_index("c")
    s = lax.axis_index("s")
    wid = s * 2 + c
    one16 = jnp.ones((16,), jnp.int32)
    z16 = jnp.zeros((16,), jnp.int32)
    for i in range(CH // 16):
        ones[pl.ds(16 * i, 16)] = one16

    @pl.loop(0, RPT // 16)
    def _zfill(i):
        zeros[pl.ds(16 * i, 16)] = z16

    pltpu.sync_copy(zeros, hist_sh.at[pl.ds(s * RPT, RPT)])
    plsc.subcore_barrier()

    @pl.loop(0, EPAD // (32 * CH))
    def _chunk(k):
        base = wid * (EPAD // 32) + k * CH
        pltpu.sync_copy(dst_hbm.at[pl.ds(base, CH)], didx)
        for j in range(NB):
            sh = jnp.zeros((16,), jnp.int32) + (10 * j)
            for i in range(CH // 16):
                dsh[pl.ds(16 * i, 16)] = didx[pl.ds(16 * i, 16)] + sh
            pltpu.sync_copy(ones, hist_sh.at[dsh], add=True)

    plsc.subcore_barrier()
    pltpu.sync_copy(hist_sh.at[pl.ds(s * RPT, RPT)], zeros)
    pltpu.sync_copy(zeros, out_hbm.at[pl.ds(c * TBL + s * RPT, RPT)])


_hist = functools.partial(
    pl.kernel,
    out_type=jax.ShapeDtypeStruct((2 * TBL,), jnp.int32),
    mesh=_mesh,
    scratch_types=[
        pltpu.VMEM((CH,), jnp.int32),
        pltpu.VMEM((CH,), jnp.int32),
        pltpu.VMEM((CH,), jnp.int32),
        pltpu.VMEM((RPT,), jnp.int32),
        pltpu.VMEM_SHARED((TBL,), jnp.int32),
    ],
)(_hist_body)


def _dense_body(x_ref, W1_ref, b1_ref, W2_ref, b2_ref, g_ref,
                be_ref, Wg_ref, hw_ref, outf_ref):
    x = x_ref[...]
    h = jax.nn.silu(x * W1_ref[...] + b1_ref[...])
    h = jax.nn.silu(
        jnp.dot(h, W2_ref[...], preferred_element_type=jnp.float32)
        + b2_ref[...])
    mu = jnp.mean(h, axis=-1, keepdims=True)
    var = jnp.mean((h - mu) ** 2, axis=-1, keepdims=True)
    hn = (h - mu) * lax.rsqrt(var + EPS_LN) * g_ref[...] + be_ref[...]
    h = jax.nn.silu(hn)
    hw = jnp.dot(h, Wg_ref[...], preferred_element_type=jnp.float32)
    hw_ref[...] = hw
    outf_ref[...] = jax.nn.relu(hw) + hw


def _dense_pass(x, W1, b1, W2, b2, ln_g, ln_b, Wg):
    grid = (N // _ROWS,)
    full = lambda i: (0, 0)
    row_spec = pl.BlockSpec((_ROWS, 1), lambda i: (i, 0))
    out_spec = pl.BlockSpec((_ROWS, D), lambda i: (i, 0))
    w_spec = pl.BlockSpec((1, D), full)
    m_spec = pl.BlockSpec((D, D), full)
    return pl.pallas_call(
        _dense_body,
        grid=grid,
        in_specs=[row_spec, w_spec, w_spec, m_spec, w_spec,
                  w_spec, w_spec, m_spec],
        out_specs=[out_spec, out_spec],
        out_shape=[jax.ShapeDtypeStruct((N, D), jnp.float32),
                   jax.ShapeDtypeStruct((N, D), jnp.float32)],
    )(x, W1, b1.reshape(1, D), W2, b2.reshape(1, D),
      ln_g.reshape(1, D), ln_b.reshape(1, D), Wg)


_UR = 1600


def _upass_body(h0_ref, h1_ref, hw_ref, u_ref, dinv_ref):
    deg = h0_ref[...] + h1_ref[...] + 1
    dinv = lax.rsqrt(deg.astype(jnp.float32))
    dinv_ref[...] = dinv
    u_ref[...] = dinv * hw_ref[...]


def _u_pass(hists_col, hw):
    grid = (TBL // _UR,)
    col = lambda i: (i, 0)
    return pl.pallas_call(
        _upass_body,
        grid=grid,
        in_specs=[pl.BlockSpec((_UR, 1), col),
                  pl.BlockSpec((_UR, 1), lambda i: (i + TBL // _UR, 0)),
                  pl.BlockSpec((_UR, D), lambda i: (i, 0))],
        out_specs=[pl.BlockSpec((_UR, D), lambda i: (i, 0)),
                   pl.BlockSpec((_UR, 1), col)],
        out_shape=[jax.ShapeDtypeStruct((TBL, D), jnp.float32),
                   jax.ShapeDtypeStruct((TBL, 1), jnp.float32)],
    )(hists_col, hists_col, hw)


BCH = 256
NG = EPAD // (16 * BCH)
GR = 2 * BCH
SB = 10
NSB = NG // SB


def _edge_scatter_body(srcp_hbm, dstp_hbm, u_hbm, out_hbm, sbig, dbig,
                       sidx0, didx0, sidx1, didx1, rows0, rows1, acc_sh,
                       sem0, sem1):
    c = lax.axis_index("c")
    s = lax.axis_index("s")
    z16 = jnp.zeros((16,), jnp.float32)
    sh0 = jnp.zeros((16,), jnp.int32) + c * 20
    sh1 = jnp.zeros((16,), jnp.int32) + (c * 20 + 10)
    ebase = s * (EPAD // 16)

    @pl.loop(0, GR)
    def _zrow(i):
        for k in range(D // 16):
            rows0[i, pl.ds(k * 16, 16)] = z16

    pltpu.sync_copy(rows0, acc_sh.at[pl.ds(s * RPT, GR)])
    pltpu.sync_copy(rows0.at[pl.ds(0, RPT - GR)],
                    acc_sh.at[pl.ds(s * RPT + GR, RPT - GR)])
    plsc.subcore_barrier()

    def _fire(k, sidx, didx, rows, sem):
        for i in range(BCH // 16):
            vs = sbig[pl.ds(k * BCH + 16 * i, 16)]
            vd = dbig[pl.ds(k * BCH + 16 * i, 16)]
            sidx[pl.ds(16 * i, 16)] = vs + sh0
            sidx[pl.ds(BCH + 16 * i, 16)] = vs + sh1
            didx[pl.ds(16 * i, 16)] = vd + sh0
            didx[pl.ds(BCH + 16 * i, 16)] = vd + sh1
        pltpu.async_copy(u_hbm.at[sidx], rows, sem)

    def _drain_scatter(didx, rows, sem):
        pltpu.make_async_copy(u_hbm.at[pl.ds(0, GR)], rows, sem).wait()
        pltpu.sync_copy(rows, acc_sh.at[didx], add=True)

    ring = [(sidx0, didx0, rows0, sem0), (sidx1, didx1, rows1, sem1)]

    @pl.loop(0, NSB)
    def _sblk(u):
        base = ebase + u * (SB * BCH)
        pltpu.sync_copy(srcp_hbm.at[pl.ds(base, SB * BCH)], sbig)
        pltpu.sync_copy(dstp_hbm.at[pl.ds(base, SB * BCH)], dbig)
        _fire(0, *ring[0])
        for k in range(1, SB):
            _fire(k, *ring[k % 2])
            _drain_scatter(*ring[(k - 1) % 2][1:])
        _drain_scatter(*ring[(SB - 1) % 2][1:])

    plsc.subcore_barrier()
    pltpu.sync_copy(acc_sh.at[pl.ds(s * RPT, GR)], rows0)
    pltpu.sync_copy(rows0, out_hbm.at[pl.ds(c * TBL + s * RPT, GR)])
    pltpu.sync_copy(acc_sh.at[pl.ds(s * RPT + GR, RPT - GR)],
                    rows0.at[pl.ds(0, RPT - GR)])
    pltpu.sync_copy(rows0.at[pl.ds(0, RPT - GR)],
                    out_hbm.at[pl.ds(c * TBL + s * RPT + GR, RPT - GR)])


_edge_scatter = functools.partial(
    pl.kernel,
    out_type=jax.ShapeDtypeStruct((2 * TBL, D), jnp.float32),
    mesh=_mesh,
    compiler_params=pltpu.CompilerParams(use_tc_tiling_on_sc=False),
    scratch_types=[
        pltpu.VMEM((SB * BCH,), jnp.int32),
        pltpu.VMEM((SB * BCH,), jnp.int32),
        pltpu.VMEM((GR,), jnp.int32),
        pltpu.VMEM((GR,), jnp.int32),
        pltpu.VMEM((GR,), jnp.int32),
        pltpu.VMEM((GR,), jnp.int32),
        pltpu.VMEM((GR, D), jnp.float32),
        pltpu.VMEM((GR, D), jnp.float32),
        pltpu.VMEM_SHARED((TBL, D), jnp.float32),
        pltpu.SemaphoreType.DMA,
        pltpu.SemaphoreType.DMA,
    ],
)(_edge_scatter_body)


def _combine_body(a0_ref, a1_ref, u_ref, dinv_ref, out_ref):
    a = a0_ref[...] + a1_ref[...]
    y = dinv_ref[...] * (u_ref[...] + a)
    out_ref[...] = jax.nn.relu(y) + y


def _combine_pass(u, acc, dinv):
    grid = (ACTIVE // 256,)
    return pl.pallas_call(
        _combine_body,
        grid=grid,
        in_specs=[pl.BlockSpec((256, D), lambda i: (i, 0)),
                  pl.BlockSpec((256, D), lambda i: (i + TBL // 256, 0)),
                  pl.BlockSpec((256, D), lambda i: (i, 0)),
                  pl.BlockSpec((256, 1), lambda i: (i, 0))],
        out_specs=pl.BlockSpec((256, D), lambda i: (i, 0)),
        out_shape=jax.ShapeDtypeStruct((ACTIVE, D), jnp.float32),
    )(acc, acc, u, dinv)


def kernel(x, go_edge_index, W1, b1, W2, b2, ln_g, ln_b, Wg):
    pad = jnp.full((EPAD - E,), SENT, dtype=jnp.int32)
    srcp = jnp.concatenate([go_edge_index[0], pad])
    dstp = jnp.concatenate([go_edge_index[1], pad])

    hists_col = _hist(dstp).reshape(2 * TBL, 1)
    hw, outf = _dense_pass(x, W1, b1, W2, b2, ln_g, ln_b, Wg)
    u, dinv = _u_pass(hists_col, hw[:TBL])
    acc = _edge_scatter(srcp, dstp, u)
    out_active = _combine_pass(u, acc, dinv)
    return jnp.concatenate([out_active, outf[ACTIVE:]], axis=0)


# speedup vs baseline: 1.3126x; 1.0508x over previous
import functools

import jax
import jax.numpy as jnp
from jax import lax
from jax.experimental import pallas as pl
from jax.experimental.pallas import tpu as pltpu
from jax.experimental.pallas import tpu_sc as plsc

N = 50000
L = 12500
NB = 4
E = 200000
D = 64
EPS_LN = 1e-5

TBL = 12800
ACTIVE = 12544
SENT = 12700
EPAD = 204800
CH = 128
RPT = TBL // 16

_ROWS = 2000

_mesh = plsc.VectorSubcoreMesh(core_axis_name="c", subcore_axis_name="s",
                               num_cores=2, num_subcores=16)


def _hist_body(dst_hbm, out_hbm, didx, dsh, ones, zeros, hist_sh):
    c = lax.axis_index("c")
    s = lax.axis_index("s")
    wid = s * 2 + c
    one16 = jnp.ones((16,), jnp.int32)
    z16 = jnp.zeros((16,), jnp.int32)
    for i in range(CH // 16):
        ones[pl.ds(16 * i, 16)] = one16

    @pl.loop(0, RPT // 16)
    def _zfill(i):
        zeros[pl.ds(16 * i, 16)] = z16

    pltpu.sync_copy(zeros, hist_sh.at[pl.ds(s * RPT, RPT)])
    plsc.subcore_barrier()

    @pl.loop(0, EPAD // (32 * CH))
    def _chunk(k):
        base = wid * (EPAD // 32) + k * CH
        pltpu.sync_copy(dst_hbm.at[pl.ds(base, CH)], didx)
        for j in range(NB):
            sh = jnp.zeros((16,), jnp.int32) + (10 * j)
            for i in range(CH // 16):
                dsh[pl.ds(16 * i, 16)] = didx[pl.ds(16 * i, 16)] + sh
            pltpu.sync_copy(ones, hist_sh.at[dsh], add=True)

    plsc.subcore_barrier()
    pltpu.sync_copy(hist_sh.at[pl.ds(s * RPT, RPT)], zeros)
    pltpu.sync_copy(zeros, out_hbm.at[pl.ds(c * TBL + s * RPT, RPT)])


_hist = functools.partial(
    pl.kernel,
    out_type=jax.ShapeDtypeStruct((2 * TBL,), jnp.int32),
    mesh=_mesh,
    scratch_types=[
        pltpu.VMEM((CH,), jnp.int32),
        pltpu.VMEM((CH,), jnp.int32),
        pltpu.VMEM((CH,), jnp.int32),
        pltpu.VMEM((RPT,), jnp.int32),
        pltpu.VMEM_SHARED((TBL,), jnp.int32),
    ],
)(_hist_body)


def _dense_body(x_ref, W1_ref, b1_ref, W2_ref, b2_ref, g_ref,
                be_ref, Wg_ref, hw_ref, outf_ref):
    x = x_ref[...]
    h = jax.nn.silu(x * W1_ref[...] + b1_ref[...])
    h = jax.nn.silu(
        jnp.dot(h, W2_ref[...], preferred_element_type=jnp.float32)
        + b2_ref[...])
    mu = jnp.mean(h, axis=-1, keepdims=True)
    var = jnp.mean((h - mu) ** 2, axis=-1, keepdims=True)
    hn = (h - mu) * lax.rsqrt(var + EPS_LN) * g_ref[...] + be_ref[...]
    h = jax.nn.silu(hn)
    hw = jnp.dot(h, Wg_ref[...], preferred_element_type=jnp.float32)
    hw_ref[...] = hw
    outf_ref[...] = jax.nn.relu(hw) + hw


def _dense_pass(x, W1, b1, W2, b2, ln_g, ln_b, Wg):
    grid = (N // _ROWS,)
    full = lambda i: (0, 0)
    row_spec = pl.BlockSpec((_ROWS, 1), lambda i: (i, 0))
    out_spec = pl.BlockSpec((_ROWS, D), lambda i: (i, 0))
    w_spec = pl.BlockSpec((1, D), full)
    m_spec = pl.BlockSpec((D, D), full)
    return pl.pallas_call(
        _dense_body,
        grid=grid,
        in_specs=[row_spec, w_spec, w_spec, m_spec, w_spec,
                  w_spec, w_spec, m_spec],
        out_specs=[out_spec, out_spec],
        out_shape=[jax.ShapeDtypeStruct((N, D), jnp.float32),
                   jax.ShapeDtypeStruct((N, D), jnp.float32)],
    )(x, W1, b1.reshape(1, D), W2, b2.reshape(1, D),
      ln_g.reshape(1, D), ln_b.reshape(1, D), Wg)


_UR = 1600


def _upass_body(h0_ref, h1_ref, hw_ref, u_ref, dinv_ref):
    deg = h0_ref[...] + h1_ref[...] + 1
    dinv = lax.rsqrt(deg.astype(jnp.float32))
    dinv_ref[...] = dinv
    u_ref[...] = dinv * hw_ref[...]


def _u_pass(hists_col, hw):
    grid = (TBL // _UR,)
    col = lambda i: (i, 0)
    return pl.pallas_call(
        _upass_body,
        grid=grid,
        in_specs=[pl.BlockSpec((_UR, 1), col),
                  pl.BlockSpec((_UR, 1), lambda i: (i + TBL // _UR, 0)),
                  pl.BlockSpec((_UR, D), lambda i: (i, 0))],
        out_specs=[pl.BlockSpec((_UR, D), lambda i: (i, 0)),
                   pl.BlockSpec((_UR, 1), col)],
        out_shape=[jax.ShapeDtypeStruct((TBL, D), jnp.float32),
                   jax.ShapeDtypeStruct((TBL, 1), jnp.float32)],
    )(hists_col, hists_col, hw)


BCH = 256
NG = EPAD // (16 * BCH)
GR = 2 * BCH
SB = 10
NSB = NG // SB


def _edge_scatter_body(srcp_hbm, dstp_hbm, u_hbm, out_hbm, sbig, dbig,
                       sidx0, didx0, sidx1, didx1, rows0, rows1, acc_sh,
                       sem0, sem1):
    c = lax.axis_index("c")
    s = lax.axis_index("s")
    z16 = jnp.zeros((16,), jnp.float32)
    sh0 = jnp.zeros((16,), jnp.int32) + c * 20
    sh1 = jnp.zeros((16,), jnp.int32) + (c * 20 + 10)
    ebase = s * (EPAD // 16)

    @pl.loop(0, GR)
    def _zrow(i):
        for k in range(D // 16):
            rows0[i, pl.ds(k * 16, 16)] = z16

    pltpu.sync_copy(rows0, acc_sh.at[pl.ds(s * RPT, GR)])
    pltpu.sync_copy(rows0.at[pl.ds(0, RPT - GR)],
                    acc_sh.at[pl.ds(s * RPT + GR, RPT - GR)])
    plsc.subcore_barrier()

    def _fire(k, sidx, didx, rows, sem):
        for i in range(BCH // 16):
            vs = sbig[pl.ds(k * BCH + 16 * i, 16)]
            vd = dbig[pl.ds(k * BCH + 16 * i, 16)]
            sidx[pl.ds(16 * i, 16)] = vs + sh0
            sidx[pl.ds(BCH + 16 * i, 16)] = vs + sh1
            didx[pl.ds(16 * i, 16)] = vd + sh0
            didx[pl.ds(BCH + 16 * i, 16)] = vd + sh1
        pltpu.async_copy(u_hbm.at[sidx], rows, sem)

    def _drain_scatter(didx, rows, sem):
        pltpu.make_async_copy(u_hbm.at[pl.ds(0, GR)], rows, sem).wait()
        pltpu.sync_copy(rows, acc_sh.at[didx], add=True)

    ring = [(sidx0, didx0, rows0, sem0), (sidx1, didx1, rows1, sem1)]

    def _load_idx(u):
        base = ebase + u * (SB * BCH)
        pltpu.sync_copy(srcp_hbm.at[pl.ds(base, SB * BCH)], sbig)
        pltpu.sync_copy(dstp_hbm.at[pl.ds(base, SB * BCH)], dbig)

    def _sblk_body(first):
        _fire(0, *ring[0])
        if not first:
            _drain_scatter(*ring[(SB - 1) % 2][1:])
        for k in range(1, SB):
            _fire(k, *ring[k % 2])
            _drain_scatter(*ring[(k - 1) % 2][1:])

    _load_idx(0)
    _sblk_body(True)

    @pl.loop(1, NSB)
    def _sblk(u):
        _load_idx(u)
        _sblk_body(False)

    _drain_scatter(*ring[(SB - 1) % 2][1:])
    plsc.subcore_barrier()
    pltpu.sync_copy(acc_sh.at[pl.ds(s * RPT, GR)], rows0)
    pltpu.sync_copy(rows0, out_hbm.at[pl.ds(c * TBL + s * RPT, GR)])
    pltpu.sync_copy(acc_sh.at[pl.ds(s * RPT + GR, RPT - GR)],
                    rows0.at[pl.ds(0, RPT - GR)])
    pltpu.sync_copy(rows0.at[pl.ds(0, RPT - GR)],
                    out_hbm.at[pl.ds(c * TBL + s * RPT + GR, RPT - GR)])


_edge_scatter = functools.partial(
    pl.kernel,
    out_type=jax.ShapeDtypeStruct((2 * TBL, D), jnp.float32),
    mesh=_mesh,
    compiler_params=pltpu.CompilerParams(use_tc_tiling_on_sc=False),
    scratch_types=[
        pltpu.VMEM((SB * BCH,), jnp.int32),
        pltpu.VMEM((SB * BCH,), jnp.int32),
        pltpu.VMEM((GR,), jnp.int32),
        pltpu.VMEM((GR,), jnp.int32),
        pltpu.VMEM((GR,), jnp.int32),
        pltpu.VMEM((GR,), jnp.int32),
        pltpu.VMEM((GR, D), jnp.float32),
        pltpu.VMEM((GR, D), jnp.float32),
        pltpu.VMEM_SHARED((TBL, D), jnp.float32),
        pltpu.SemaphoreType.DMA,
        pltpu.SemaphoreType.DMA,
    ],
)(_edge_scatter_body)


def _combine_body(a0_ref, a1_ref, u_ref, dinv_ref, out_ref):
    a = a0_ref[...] + a1_ref[...]
    y = dinv_ref[...] * (u_ref[...] + a)
    out_ref[...] = jax.nn.relu(y) + y


def _combine_pass(u, acc, dinv):
    grid = (ACTIVE // 256,)
    return pl.pallas_call(
        _combine_body,
        grid=grid,
        in_specs=[pl.BlockSpec((256, D), lambda i: (i, 0)),
                  pl.BlockSpec((256, D), lambda i: (i + TBL // 256, 0)),
                  pl.BlockSpec((256, D), lambda i: (i, 0)),
                  pl.BlockSpec((256, 1), lambda i: (i, 0))],
        out_specs=pl.BlockSpec((256, D), lambda i: (i, 0)),
        out_shape=jax.ShapeDtypeStruct((ACTIVE, D), jnp.float32),
    )(acc, acc, u, dinv)


def kernel(x, go_edge_index, W1, b1, W2, b2, ln_g, ln_b, Wg):
    pad = jnp.full((EPAD - E,), SENT, dtype=jnp.int32)
    srcp = jnp.concatenate([go_edge_index[0], pad])
    dstp = jnp.concatenate([go_edge_index[1], pad])

    hists_col = _hist(dstp).reshape(2 * TBL, 1)
    hw, outf = _dense_pass(x, W1, b1, W2, b2, ln_g, ln_b, Wg)
    u, dinv = _u_pass(hists_col, hw[:TBL])
    acc = _edge_scatter(srcp, dstp, u)
    out_active = _combine_pass(u, acc, dinv)
    return jnp.concatenate([out_active, outf[ACTIVE:]], axis=0)
